# baseline (speedup 1.0000x reference)
import functools

import jax
import jax.numpy as jnp
from jax import lax
from jax.experimental import pallas as pl
from jax.experimental.pallas import tpu as pltpu
from jax.experimental.pallas import tpu_sc as plsc

B = 4096
EMB = 64
HID = 64
NCLS = 1000
ROWS_PER_TILE = 8


def _make_sc_gather(T: int, Bn: int):
    info = plsc.get_sparse_core_info()
    NC, NS = info.num_cores, info.num_subcores
    NW = NC * NS
    assert Bn % NW == 0
    b_per_w = Bn // NW
    mesh = plsc.VectorSubcoreMesh(core_axis_name="c", subcore_axis_name="s")

    @functools.partial(
        pl.kernel,
        mesh=mesh,
        out_type=jax.ShapeDtypeStruct((Bn, ROWS_PER_TILE, EMB), jnp.float32),
        scratch_types=[
            pltpu.VMEM((b_per_w,), jnp.int32),
            pltpu.SemaphoreType.DMA,
        ],
    )
    def gather_kernel(table_hbm, idx_hbm, out_hbm, idx_s, sem):
        wid = lax.axis_index("s") * NC + lax.axis_index("c")
        base = wid * b_per_w
        pltpu.sync_copy(idx_hbm.at[pl.ds(base, b_per_w)], idx_s)
        descs = []
        for g in range(b_per_w // 16):
            tv = idx_s[pl.ds(g * 16, 16)]
            for k in range(16):
                t = tv[k]
                descs.append(
                    pltpu.async_copy(
                        table_hbm.at[pl.ds(t, 1)],
                        out_hbm.at[pl.ds(base + g * 16 + k, 1)],
                        sem,
                    )
                )
        for d in descs:
            d.wait()

    return gather_kernel


def _mlp_body(tiles_ref, sub_ref, w1_ref, b1_ref, w2_ref, b2_ref, out_ref):
    sub = sub_ref[...]
    onehot = (
        sub == lax.broadcasted_iota(jnp.int32, (1, ROWS_PER_TILE, 1), 1)
    ).astype(jnp.float32)
    pooled = jnp.sum(tiles_ref[...] * onehot, axis=1)
    h = jnp.maximum(
        jnp.dot(pooled, w1_ref[...], preferred_element_type=jnp.float32)
        + b1_ref[...],
        0.0,
    )
    out_ref[...] = (
        jnp.dot(h, w2_ref[...], preferred_element_type=jnp.float32) + b2_ref[...]
    )


def _mlp(tiles, sub, W1, b1, W2, b2):
    BLK = 512
    grid = (B // BLK,)
    return pl.pallas_call(
        _mlp_body,
        grid=grid,
        in_specs=[
            pl.BlockSpec((BLK, ROWS_PER_TILE, EMB), lambda i: (i, 0, 0)),
            pl.BlockSpec((BLK, 1, 1), lambda i: (i, 0, 0)),
            pl.BlockSpec((EMB, HID), lambda i: (0, 0)),
            pl.BlockSpec((1, HID), lambda i: (0, 0)),
            pl.BlockSpec((HID, NCLS), lambda i: (0, 0)),
            pl.BlockSpec((1, NCLS), lambda i: (0, 0)),
        ],
        out_specs=pl.BlockSpec((BLK, NCLS), lambda i: (i, 0)),
        out_shape=jax.ShapeDtypeStruct((B, NCLS), jnp.float32),
    )(tiles, sub, W1, b1.reshape(1, HID), W2, b2.reshape(1, NCLS))


def kernel(input_ids, offsets, table, W1, b1, W2, b2):
    del offsets
    ids = input_ids.astype(jnp.int32)
    tile_idx = ids // ROWS_PER_TILE
    sub_idx = (ids % ROWS_PER_TILE).reshape(B, 1, 1)
    table3 = table.reshape(
        table.shape[0] // ROWS_PER_TILE, ROWS_PER_TILE, EMB
    )
    gather = _make_sc_gather(table3.shape[0], B)
    tiles = gather(table3, tile_idx)
    return _mlp(tiles, sub_idx, W1, b1, W2, b2)


# speedup vs baseline: 1.1790x; 1.1790x over previous
import functools

import jax
import jax.numpy as jnp
from jax import lax
from jax.experimental import pallas as pl
from jax.experimental.pallas import tpu as pltpu
from jax.experimental.pallas import tpu_sc as plsc

B = 4096
EMB = 64
HID = 64
NCLS = 1000
PAIR = 2


def _make_sc_gather(T: int, Bn: int):
    info = plsc.get_sparse_core_info()
    NC, NS = info.num_cores, info.num_subcores
    NW = NC * NS
    assert Bn % NW == 0
    b_per_w = Bn // NW
    mesh = plsc.VectorSubcoreMesh(core_axis_name="c", subcore_axis_name="s")

    @functools.partial(
        pl.kernel,
        mesh=mesh,
        out_type=jax.ShapeDtypeStruct((Bn, PAIR * EMB), jnp.float32),
        scratch_types=[
            pltpu.VMEM((b_per_w,), jnp.int32),
            pltpu.VMEM((b_per_w, PAIR * EMB), jnp.float32),
            pltpu.SemaphoreType.DMA,
        ],
    )
    def gather_kernel(table_hbm, idx_hbm, out_hbm, idx_v, rows_v, sem):
        wid = lax.axis_index("s") * NC + lax.axis_index("c")
        base = wid * b_per_w
        pltpu.sync_copy(idx_hbm.at[pl.ds(base, b_per_w)], idx_v)
        pltpu.async_copy(table_hbm.at[idx_v], rows_v, sem).wait()
        pltpu.sync_copy(rows_v, out_hbm.at[pl.ds(base, b_per_w)])

    return gather_kernel


def _mlp_body(pairs_ref, sub_ref, w1_ref, b1_ref, w2_ref, b2_ref, out_ref):
    sub = sub_ref[...]
    onehot = (
        sub == lax.broadcasted_iota(jnp.int32, (1, PAIR, 1), 1)
    ).astype(jnp.float32)
    pooled = jnp.sum(pairs_ref[...] * onehot, axis=1)
    h = jnp.maximum(
        jnp.dot(pooled, w1_ref[...], preferred_element_type=jnp.float32)
        + b1_ref[...],
        0.0,
    )
    out_ref[...] = (
        jnp.dot(h, w2_ref[...], preferred_element_type=jnp.float32) + b2_ref[...]
    )


def _mlp(pairs, sub, W1, b1, W2, b2):
    BLK = 512
    grid = (B // BLK,)
    return pl.pallas_call(
        _mlp_body,
        grid=grid,
        in_specs=[
            pl.BlockSpec((BLK, PAIR, EMB), lambda i: (i, 0, 0)),
            pl.BlockSpec((BLK, 1, 1), lambda i: (i, 0, 0)),
            pl.BlockSpec((EMB, HID), lambda i: (0, 0)),
            pl.BlockSpec((1, HID), lambda i: (0, 0)),
            pl.BlockSpec((HID, NCLS), lambda i: (0, 0)),
            pl.BlockSpec((1, NCLS), lambda i: (0, 0)),
        ],
        out_specs=pl.BlockSpec((BLK, NCLS), lambda i: (i, 0)),
        out_shape=jax.ShapeDtypeStruct((B, NCLS), jnp.float32),
    )(pairs, sub, W1, b1.reshape(1, HID), W2, b2.reshape(1, NCLS))


def kernel(input_ids, offsets, table, W1, b1, W2, b2):
    del offsets
    ids = input_ids.astype(jnp.int32)
    pair_idx = ids // PAIR
    sub_idx = (ids % PAIR).reshape(B, 1, 1)
    table2 = table.reshape(table.shape[0] // PAIR, PAIR * EMB)
    gather = _make_sc_gather(table2.shape[0], B)
    pairs = gather(table2, pair_idx).reshape(B, PAIR, EMB)
    return _mlp(pairs, sub_idx, W1, b1, W2, b2)


# speedup vs baseline: 1.9238x; 1.6317x over previous
import functools

import jax
import jax.numpy as jnp
from jax import lax
from jax.experimental import pallas as pl
from jax.experimental.pallas import tpu as pltpu
from jax.experimental.pallas import tpu_sc as plsc

B = 4096
EMB = 64
HID = 64
NCLS = 1000
SUB = 8


def _make_sc_gather(V: int, Bn: int):
    info = plsc.get_sparse_core_info()
    NC, NS = info.num_cores, info.num_subcores
    NW = NC * NS
    assert Bn % NW == 0
    b_per_w = Bn // NW
    CH = 64
    n_ch = b_per_w // CH
    mesh = plsc.VectorSubcoreMesh(core_axis_name="c", subcore_axis_name="s")

    @functools.partial(
        pl.kernel,
        mesh=mesh,
        out_type=jax.ShapeDtypeStruct((Bn, SUB, EMB), jnp.float32),
        scratch_types=[
            pltpu.VMEM((b_per_w,), jnp.int32),
            pltpu.VMEM((CH, SUB, EMB), jnp.float32),
            pltpu.SemaphoreType.DMA,
        ],
    )
    def gather_kernel(table_hbm, idx_hbm, out_hbm, idx_v, stage_v, sem):
        wid = lax.axis_index("s") * NC + lax.axis_index("c")
        base = wid * b_per_w
        pltpu.sync_copy(idx_hbm.at[pl.ds(base, b_per_w)], idx_v)
        for ch in range(n_ch):
            descs = []
            for g in range(CH // 16):
                tv = idx_v[pl.ds(ch * CH + g * 16, 16)]
                tal = (tv >> 3) << 3
                for k in range(16):
                    descs.append(
                        pltpu.async_copy(
                            table_hbm.at[pl.ds(pl.multiple_of(tal[k], SUB), SUB)],
                            stage_v.at[g * 16 + k],
                            sem,
                        )
                    )
            for d in descs:
                d.wait()
            pltpu.sync_copy(stage_v, out_hbm.at[pl.ds(base + ch * CH, CH)])

    return gather_kernel


def _mlp_body(tiles_ref, ids_ref, w1_ref, b1_ref, w2_ref, b2_ref, out_ref):
    sub = ids_ref[...] & (SUB - 1)
    onehot = (
        sub == lax.broadcasted_iota(jnp.int32, (1, SUB, 1), 1)
    ).astype(jnp.float32)
    pooled = jnp.sum(tiles_ref[...] * onehot, axis=1)
    h = jnp.maximum(
        jnp.dot(pooled, w1_ref[...], preferred_element_type=jnp.float32)
        + b1_ref[...],
        0.0,
    )
    out_ref[...] = (
        jnp.dot(h, w2_ref[...], preferred_element_type=jnp.float32) + b2_ref[...]
    )


def _mlp(tiles, ids, W1, b1, W2, b2):
    BLK = 512
    grid = (B // BLK,)
    return pl.pallas_call(
        _mlp_body,
        grid=grid,
        in_specs=[
            pl.BlockSpec((BLK, SUB, EMB), lambda i: (i, 0, 0)),
            pl.BlockSpec((BLK, 1, 1), lambda i: (i, 0, 0)),
            pl.BlockSpec((EMB, HID), lambda i: (0, 0)),
            pl.BlockSpec((1, HID), lambda i: (0, 0)),
            pl.BlockSpec((HID, NCLS), lambda i: (0, 0)),
            pl.BlockSpec((1, NCLS), lambda i: (0, 0)),
        ],
        out_specs=pl.BlockSpec((BLK, NCLS), lambda i: (i, 0)),
        out_shape=jax.ShapeDtypeStruct((B, NCLS), jnp.float32),
    )(tiles, ids, W1, b1.reshape(1, HID), W2, b2.reshape(1, NCLS))


def kernel(input_ids, offsets, table, W1, b1, W2, b2):
    del offsets
    ids = input_ids.astype(jnp.int32)
    gather = _make_sc_gather(table.shape[0], B)
    tiles = gather(table, ids)
    return _mlp(tiles, ids.reshape(B, 1, 1), W1, b1, W2, b2)


# speedup vs baseline: 2.7229x; 1.4154x over previous
import functools

import jax
import jax.numpy as jnp
from jax import lax
from jax.experimental import pallas as pl
from jax.experimental.pallas import tpu as pltpu
from jax.experimental.pallas import tpu_sc as plsc

B = 4096
EMB = 64
HID = 64
NCLS = 1000
SUB = 8


def _make_sc_gather(V: int, Bn: int):
    info = plsc.get_sparse_core_info()
    NC, NS = info.num_cores, info.num_subcores
    NW = NC * NS
    assert Bn % NW == 0
    b_per_w = Bn // NW
    CH = 64
    n_ch = b_per_w // CH
    mesh = plsc.VectorSubcoreMesh(core_axis_name="c", subcore_axis_name="s")

    @functools.partial(
        pl.kernel,
        mesh=mesh,
        out_type=jax.ShapeDtypeStruct((Bn, SUB, EMB), jnp.float32),
        scratch_types=[
            pltpu.VMEM((b_per_w,), jnp.int32),
            pltpu.VMEM((CH, SUB, EMB), jnp.float32),
            pltpu.SemaphoreType.DMA,
        ],
    )
    def gather_kernel(table_hbm, idx_hbm, out_hbm, idx_v, stage_v, sem):
        wid = lax.axis_index("s") * NC + lax.axis_index("c")
        base = wid * b_per_w
        pltpu.sync_copy(idx_hbm.at[pl.ds(base, b_per_w)], idx_v)
        for ch in range(n_ch):
            descs = []
            for g in range(CH // 16):
                tv = idx_v[pl.ds(ch * CH + g * 16, 16)]
                tiles = tv >> 3
                for k in range(16):
                    descs.append(
                        pltpu.async_copy(
                            table_hbm.at[pl.ds(tiles[k], 1)],
                            stage_v.at[pl.ds(g * 16 + k, 1)],
                            sem,
                        )
                    )
            for d in descs:
                d.wait()
            pltpu.sync_copy(stage_v, out_hbm.at[pl.ds(base + ch * CH, CH)])

    return gather_kernel


def _mlp_body(tiles_ref, ids_ref, w1_ref, b1_ref, w2_ref, b2_ref, out_ref):
    sub = ids_ref[...] & (SUB - 1)
    onehot = (
        sub == lax.broadcasted_iota(jnp.int32, (1, SUB, 1), 1)
    ).astype(jnp.float32)
    pooled = jnp.sum(tiles_ref[...] * onehot, axis=1)
    h = jnp.maximum(
        jnp.dot(pooled, w1_ref[...], preferred_element_type=jnp.float32)
        + b1_ref[...],
        0.0,
    )
    out_ref[...] = (
        jnp.dot(h, w2_ref[...], preferred_element_type=jnp.float32) + b2_ref[...]
    )


def _mlp(tiles, ids, W1, b1, W2, b2):
    BLK = 512
    grid = (B // BLK,)
    return pl.pallas_call(
        _mlp_body,
        grid=grid,
        in_specs=[
            pl.BlockSpec((BLK, SUB, EMB), lambda i: (i, 0, 0)),
            pl.BlockSpec((BLK, 1, 1), lambda i: (i, 0, 0)),
            pl.BlockSpec((EMB, HID), lambda i: (0, 0)),
            pl.BlockSpec((1, HID), lambda i: (0, 0)),
            pl.BlockSpec((HID, NCLS), lambda i: (0, 0)),
            pl.BlockSpec((1, NCLS), lambda i: (0, 0)),
        ],
        out_specs=pl.BlockSpec((BLK, NCLS), lambda i: (i, 0)),
        out_shape=jax.ShapeDtypeStruct((B, NCLS), jnp.float32),
    )(tiles, ids, W1, b1.reshape(1, HID), W2, b2.reshape(1, NCLS))


def kernel(input_ids, offsets, table, W1, b1, W2, b2):
    del offsets
    ids = input_ids.astype(jnp.int32)
    table3 = table.reshape(table.shape[0] // SUB, SUB, EMB)
    gather = _make_sc_gather(table3.shape[0], B)
    tiles = gather(table3, ids)
    return _mlp(tiles, ids.reshape(B, 1, 1), W1, b1, W2, b2)


# speedup vs baseline: 2.9128x; 1.0697x over previous
import functools

import jax
import jax.numpy as jnp
from jax import lax
from jax.experimental import pallas as pl
from jax.experimental.pallas import tpu as pltpu
from jax.experimental.pallas import tpu_sc as plsc

B = 4096
EMB = 64
HID = 64
NCLS = 1000
SUB = 8


def _make_sc_gather(V: int, Bn: int):
    info = plsc.get_sparse_core_info()
    NC, NS = info.num_cores, info.num_subcores
    NW = NC * NS
    assert Bn % NW == 0
    b_per_w = Bn // NW
    CH = 64
    n_ch = b_per_w // CH
    mesh = plsc.VectorSubcoreMesh(core_axis_name="c", subcore_axis_name="s")

    @functools.partial(
        pl.kernel,
        mesh=mesh,
        out_type=jax.ShapeDtypeStruct((Bn, SUB, EMB), jnp.float32),
        scratch_types=[
            pltpu.VMEM((b_per_w,), jnp.int32),
            pltpu.VMEM((CH, SUB, EMB), jnp.float32),
            pltpu.SemaphoreType.DMA,
        ],
    )
    def gather_kernel(table_hbm, idx_hbm, out_hbm, idx_v, stage_v, sem):
        wid = lax.axis_index("s") * NC + lax.axis_index("c")
        base = wid * b_per_w
        pltpu.sync_copy(idx_hbm.at[pl.ds(base, b_per_w)], idx_v)
        for ch in range(n_ch):
            descs = []
            for g in range(CH // 16):
                tv = idx_v[pl.ds(ch * CH + g * 16, 16)]
                tiles = tv >> 3
                for k in range(16):
                    descs.append(
                        pltpu.async_copy(
                            table_hbm.at[pl.ds(tiles[k], 1)],
                            stage_v.at[pl.ds(g * 16 + k, 1)],
                            sem,
                        )
                    )
            for d in descs:
                d.wait()
            pltpu.sync_copy(stage_v, out_hbm.at[pl.ds(base + ch * CH, CH)])

    return gather_kernel


def _mlp_body(tiles_ref, ids_ref, w1_ref, b1_ref, w2_ref, b2_ref, out_ref):
    sub = ids_ref[...] & (SUB - 1)
    onehot = (
        sub == lax.broadcasted_iota(jnp.int32, (1, SUB, 1), 1)
    ).astype(jnp.float32)
    pooled = jnp.sum(tiles_ref[...] * onehot, axis=1)
    h = jnp.maximum(
        jnp.dot(pooled, w1_ref[...], preferred_element_type=jnp.float32)
        + b1_ref[...],
        0.0,
    )
    out_ref[...] = (
        lax.dot_general(
            w2_ref[...], h, (((0,), (1,)), ((), ())),
            preferred_element_type=jnp.float32,
        )
        + b2_ref[...]
    )


def _mlp(tiles, ids, W1, b1, W2, b2):
    BLK = 1024
    grid = (B // BLK,)
    return pl.pallas_call(
        _mlp_body,
        grid=grid,
        in_specs=[
            pl.BlockSpec((BLK, SUB, EMB), lambda i: (i, 0, 0)),
            pl.BlockSpec((BLK, 1, 1), lambda i: (i, 0, 0)),
            pl.BlockSpec((EMB, HID), lambda i: (0, 0)),
            pl.BlockSpec((1, HID), lambda i: (0, 0)),
            pl.BlockSpec((HID, NCLS), lambda i: (0, 0)),
            pl.BlockSpec((NCLS, 1), lambda i: (0, 0)),
        ],
        out_specs=pl.BlockSpec((NCLS, BLK), lambda i: (0, i)),
        out_shape=jax.ShapeDtypeStruct((NCLS, B), jnp.float32),
    )(tiles, ids, W1, b1.reshape(1, HID), W2, b2.reshape(NCLS, 1))


def kernel(input_ids, offsets, table, W1, b1, W2, b2):
    del offsets
    ids = input_ids.astype(jnp.int32)
    table3 = table.reshape(table.shape[0] // SUB, SUB, EMB)
    gather = _make_sc_gather(table3.shape[0], B)
    tiles = gather(table3, ids)
    return _mlp(tiles, ids.reshape(B, 1, 1), W1, b1, W2, b2).T


# speedup vs baseline: 7.5112x; 2.5787x over previous
import functools

import jax
import jax.numpy as jnp
from jax import lax
from jax.experimental import pallas as pl
from jax.experimental.pallas import tpu as pltpu
from jax.experimental.pallas import tpu_sc as plsc

B = 4096
EMB = 64
HID = 64
NCLS = 1000
LANES = 128


def _make_sc_gather(V: int, Bn: int):
    info = plsc.get_sparse_core_info()
    NC, NS = info.num_cores, info.num_subcores
    NW = NC * NS
    assert Bn % NW == 0
    b_per_w = Bn // NW
    RT = 8
    n_rounds = b_per_w // RT
    mesh = plsc.VectorSubcoreMesh(core_axis_name="c", subcore_axis_name="s")

    @functools.partial(
        pl.kernel,
        mesh=mesh,
        out_type=jax.ShapeDtypeStruct((EMB, Bn), jnp.float32),
        scratch_types=[
            pltpu.VMEM((b_per_w,), jnp.int32),
            pltpu.VMEM((EMB, RT * LANES), jnp.float32),
            pltpu.VMEM((EMB, b_per_w), jnp.float32),
            pltpu.SemaphoreType.DMA,
        ],
        compiler_params=pltpu.CompilerParams(needs_layout_passes=False),
    )
    def gather_kernel(tt_hbm, idx_hbm, out_hbm, idx_v, stg_v, po_v, sem):
        wid = lax.axis_index("s") * NC + lax.axis_index("c")
        base = wid * b_per_w
        pltpu.sync_copy(idx_hbm.at[pl.ds(base, b_per_w)], idx_v)
        iota = lax.iota(jnp.int32, 16)
        for r in range(n_rounds):
            idv = idx_v[pl.ds((r * RT // 16) * 16, 16)]
            descs = []
            for t in range(RT):
                j = r * RT + t
                lane_in_vec = j % 16
                cid = idv[lane_in_vec] >> 7
                descs.append(
                    pltpu.async_copy(
                        tt_hbm.at[
                            :, pl.ds(pl.multiple_of(cid * LANES, LANES), LANES)
                        ],
                        stg_v.at[:, pl.ds(t * LANES, LANES)],
                        sem,
                    )
                )
            for d in descs:
                d.wait()
            for t in range(RT):
                j = r * RT + t
                lane_in_vec = j % 16
                lane = (idv[lane_in_vec] & (LANES - 1)) + t * LANES
                lane_vec = jnp.full((16,), lane, jnp.int32)
                col_vec = jnp.full((16,), j, jnp.int32)
                for c in range(EMB // 16):
                    feat = c * 16 + iota
                    x = plsc.load_gather(stg_v, [feat, lane_vec])
                    plsc.store_scatter(po_v, [feat, col_vec], x)
        pltpu.sync_copy(po_v, out_hbm.at[:, pl.ds(base, b_per_w)])

    return gather_kernel


def _mlp_body(pt_ref, w1_ref, b1t_ref, w2_ref, b2t_ref, out_ref):
    h = jnp.maximum(
        lax.dot_general(
            w1_ref[...], pt_ref[...], (((0,), (0,)), ((), ())),
            preferred_element_type=jnp.float32,
        )
        + b1t_ref[...],
        0.0,
    )
    out_ref[...] = (
        lax.dot_general(
            w2_ref[...], h, (((0,), (0,)), ((), ())),
            preferred_element_type=jnp.float32,
        )
        + b2t_ref[...]
    )


def _mlp(pooledT, W1, b1, W2, b2):
    BLK = 1024
    grid = (B // BLK,)
    return pl.pallas_call(
        _mlp_body,
        grid=grid,
        in_specs=[
            pl.BlockSpec((EMB, BLK), lambda i: (0, i)),
            pl.BlockSpec((EMB, HID), lambda i: (0, 0)),
            pl.BlockSpec((HID, 1), lambda i: (0, 0)),
            pl.BlockSpec((HID, NCLS), lambda i: (0, 0)),
            pl.BlockSpec((NCLS, 1), lambda i: (0, 0)),
        ],
        out_specs=pl.BlockSpec((NCLS, BLK), lambda i: (0, i)),
        out_shape=jax.ShapeDtypeStruct((NCLS, B), jnp.float32),
    )(pooledT, W1, b1.reshape(HID, 1), W2, b2.reshape(NCLS, 1))


def kernel(input_ids, offsets, table, W1, b1, W2, b2):
    del offsets
    ids = input_ids.astype(jnp.int32)
    tt = table.T
    gather = _make_sc_gather(tt.shape[1], B)
    pooledT = gather(tt, ids)
    return _mlp(pooledT, W1, b1, W2, b2).T


# speedup vs baseline: 7.5460x; 1.0046x over previous
import functools

import jax
import jax.numpy as jnp
from jax import lax
from jax.experimental import pallas as pl
from jax.experimental.pallas import tpu as pltpu
from jax.experimental.pallas import tpu_sc as plsc

B = 4096
EMB = 64
HID = 64
NCLS = 1000
LANES = 128


def _make_sc_gather(V: int, Bn: int):
    info = plsc.get_sparse_core_info()
    NC, NS = info.num_cores, info.num_subcores
    NW = NC * NS
    assert Bn % NW == 0
    b_per_w = Bn // NW
    RT = 8
    n_rounds = b_per_w // RT
    mesh = plsc.VectorSubcoreMesh(core_axis_name="c", subcore_axis_name="s")

    @functools.partial(
        pl.kernel,
        mesh=mesh,
        out_type=jax.ShapeDtypeStruct((EMB, Bn), jnp.float32),
        scratch_types=[
            pltpu.VMEM((b_per_w,), jnp.int32),
            pltpu.VMEM((EMB, RT * LANES), jnp.float32),
            pltpu.VMEM((EMB, b_per_w), jnp.float32),
            pltpu.SemaphoreType.DMA,
        ],
        compiler_params=pltpu.CompilerParams(needs_layout_passes=False),
    )
    def gather_kernel(tt_hbm, idx_hbm, out_hbm, idx_v, stg_v, po_v, sem):
        wid = lax.axis_index("s") * NC + lax.axis_index("c")
        base = wid * b_per_w
        pltpu.sync_copy(idx_hbm.at[pl.ds(base, b_per_w)], idx_v)
        iota = lax.iota(jnp.int32, 16)
        for r in range(n_rounds):
            idv = idx_v[pl.ds((r * RT // 16) * 16, 16)]
            descs = []
            for t in range(RT):
                j = r * RT + t
                lane_in_vec = j % 16
                cid = idv[lane_in_vec] >> 7
                descs.append(
                    pltpu.async_copy(
                        tt_hbm.at[
                            :, pl.ds(pl.multiple_of(cid * LANES, LANES), LANES)
                        ],
                        stg_v.at[:, pl.ds(t * LANES, LANES)],
                        sem,
                    )
                )
            for d in descs:
                d.wait()
            for t in range(RT):
                j = r * RT + t
                lane_in_vec = j % 16
                lane = (idv[lane_in_vec] & (LANES - 1)) + t * LANES
                lane_vec = jnp.full((16,), lane, jnp.int32)
                col_vec = jnp.full((16,), j, jnp.int32)
                for c in range(EMB // 16):
                    feat = c * 16 + iota
                    x = plsc.load_gather(stg_v, [feat, lane_vec])
                    plsc.store_scatter(po_v, [feat, col_vec], x)
        pltpu.sync_copy(po_v, out_hbm.at[:, pl.ds(base, b_per_w)])

    return gather_kernel


def _mlp_body(pt_ref, w1_ref, b1t_ref, w2_ref, b2t_ref, out_ref):
    h = jnp.maximum(
        lax.dot_general(
            w1_ref[...], pt_ref[...], (((0,), (0,)), ((), ())),
            preferred_element_type=jnp.float32,
        )
        + b1t_ref[...],
        0.0,
    )
    out_ref[...] = (
        lax.dot_general(
            w2_ref[...], h, (((0,), (0,)), ((), ())),
            preferred_element_type=jnp.float32,
        )
        + b2t_ref[...]
    )


def _mlp(pooledT, W1, b1, W2, b2):
    BLK = 2048
    grid = (B // BLK,)
    return pl.pallas_call(
        _mlp_body,
        grid=grid,
        in_specs=[
            pl.BlockSpec((EMB, BLK), lambda i: (0, i)),
            pl.BlockSpec((EMB, HID), lambda i: (0, 0)),
            pl.BlockSpec((HID, 1), lambda i: (0, 0)),
            pl.BlockSpec((HID, NCLS), lambda i: (0, 0)),
            pl.BlockSpec((NCLS, 1), lambda i: (0, 0)),
        ],
        out_specs=pl.BlockSpec((NCLS, BLK), lambda i: (0, i)),
        out_shape=jax.ShapeDtypeStruct((NCLS, B), jnp.float32),
    )(pooledT, W1, b1.reshape(HID, 1), W2, b2.reshape(NCLS, 1))


def kernel(input_ids, offsets, table, W1, b1, W2, b2):
    del offsets
    ids = input_ids.astype(jnp.int32)
    tt = table.T
    gather = _make_sc_gather(tt.shape[1], B)
    pooledT = gather(tt, ids)
    return _mlp(pooledT, W1, b1, W2, b2).T


# speedup vs baseline: 9.1861x; 1.2174x over previous
import functools

import jax
import jax.numpy as jnp
from jax import lax
from jax.experimental import pallas as pl
from jax.experimental.pallas import tpu as pltpu
from jax.experimental.pallas import tpu_sc as plsc

B = 4096
EMB = 64
HID = 64
NCLS = 1000
LANES = 128


def _make_sc_gather(V: int, Bn: int):
    info = plsc.get_sparse_core_info()
    NC, NS = info.num_cores, info.num_subcores
    NW = NC * NS
    assert Bn % NW == 0
    b_per_w = Bn // NW
    RT = 4
    n_rounds = b_per_w // RT
    mesh = plsc.VectorSubcoreMesh(core_axis_name="c", subcore_axis_name="s")

    @functools.partial(
        pl.kernel,
        mesh=mesh,
        out_type=jax.ShapeDtypeStruct((EMB, Bn), jnp.float32),
        scratch_types=[
            pltpu.VMEM((b_per_w,), jnp.int32),
            pltpu.VMEM((EMB, RT * LANES), jnp.float32),
            pltpu.VMEM((EMB, RT * LANES), jnp.float32),
            pltpu.VMEM((EMB, b_per_w), jnp.float32),
            pltpu.SemaphoreType.DMA,
        ],
        compiler_params=pltpu.CompilerParams(needs_layout_passes=False),
    )
    def gather_kernel(tt_hbm, idx_hbm, out_hbm, idx_v, stg0_v, stg1_v, po_v, sem):
        wid = lax.axis_index("s") * NC + lax.axis_index("c")
        base = wid * b_per_w
        pltpu.sync_copy(idx_hbm.at[pl.ds(base, b_per_w)], idx_v)
        iota = lax.iota(jnp.int32, 16)
        stgs = [stg0_v, stg1_v]

        def fetch(r):
            idv = idx_v[pl.ds((r * RT // 16) * 16, 16)]
            stg = stgs[r % 2]
            descs = []
            for t in range(RT):
                j = r * RT + t
                cid = idv[j % 16] >> 7
                descs.append(
                    pltpu.async_copy(
                        tt_hbm.at[
                            :, pl.ds(pl.multiple_of(cid * LANES, LANES), LANES)
                        ],
                        stg.at[:, pl.ds(t * LANES, LANES)],
                        sem,
                    )
                )
            return descs

        def extract(r, descs):
            for d in descs:
                d.wait()
            idv = idx_v[pl.ds((r * RT // 16) * 16, 16)]
            stg = stgs[r % 2]
            for t in range(RT):
                j = r * RT + t
                lane = (idv[j % 16] & (LANES - 1)) + t * LANES
                lane_vec = jnp.full((16,), lane, jnp.int32)
                col_vec = jnp.full((16,), j, jnp.int32)
                for c in range(EMB // 16):
                    feat = c * 16 + iota
                    x = plsc.load_gather(stg, [feat, lane_vec])
                    plsc.store_scatter(po_v, [feat, col_vec], x)

        pend = fetch(0)
        for r in range(1, n_rounds):
            nxt = fetch(r)
            extract(r - 1, pend)
            pend = nxt
        extract(n_rounds - 1, pend)
        pltpu.sync_copy(po_v, out_hbm.at[:, pl.ds(base, b_per_w)])

    return gather_kernel


def _mlp_body(pt_ref, w1_ref, b1t_ref, w2_ref, b2t_ref, out_ref):
    h = jnp.maximum(
        lax.dot_general(
            w1_ref[...], pt_ref[...], (((0,), (0,)), ((), ())),
            preferred_element_type=jnp.float32,
        )
        + b1t_ref[...],
        0.0,
    )
    out_ref[...] = (
        lax.dot_general(
            w2_ref[...], h, (((0,), (0,)), ((), ())),
            preferred_element_type=jnp.float32,
        )
        + b2t_ref[...]
    )


def _mlp(pooledT, W1, b1, W2, b2):
    BLK = 2048
    grid = (B // BLK,)
    return pl.pallas_call(
        _mlp_body,
        grid=grid,
        in_specs=[
            pl.BlockSpec((EMB, BLK), lambda i: (0, i)),
            pl.BlockSpec((EMB, HID), lambda i: (0, 0)),
            pl.BlockSpec((HID, 1), lambda i: (0, 0)),
            pl.BlockSpec((HID, NCLS), lambda i: (0, 0)),
            pl.BlockSpec((NCLS, 1), lambda i: (0, 0)),
        ],
        out_specs=pl.BlockSpec((NCLS, BLK), lambda i: (0, i)),
        out_shape=jax.ShapeDtypeStruct((NCLS, B), jnp.float32),
    )(pooledT, W1, b1.reshape(HID, 1), W2, b2.reshape(NCLS, 1))


def kernel(input_ids, offsets, table, W1, b1, W2, b2):
    del offsets
    ids = input_ids.astype(jnp.int32)
    tt = table.T
    gather = _make_sc_gather(tt.shape[1], B)
    pooledT = gather(tt, ids)
    return _mlp(pooledT, W1, b1, W2, b2).T
